# baseline (speedup 1.0000x reference)
import functools

import jax
import jax.numpy as jnp
from jax import lax
from jax.experimental import pallas as pl
from jax.experimental.pallas import tpu as pltpu
from jax.experimental.pallas import tpu_sc as plsc

BATCH = 4096
SEQ = 200
HIDDEN = 64
B = BATCH * SEQ
NC = 2
NS = 16
NW = NC * NS
B_PER_W = B // NW
IDX_ROW = 128
ROWS_PER_BLOCK = 512
GPB = ROWS_PER_BLOCK // IDX_ROW
NBLK = B_PER_W // ROWS_PER_BLOCK
IDS_ROWS_PER_W = B_PER_W // IDX_ROW


def kernel(input_ids, embed):
    ids2d = input_ids.reshape(B // IDX_ROW, IDX_ROW).astype(jnp.int32)
    mesh = plsc.VectorSubcoreMesh(core_axis_name="c", subcore_axis_name="s")

    @functools.partial(
        pl.kernel,
        mesh=mesh,
        out_type=jax.ShapeDtypeStruct((B, HIDDEN), jnp.float32),
        compiler_params=pltpu.CompilerParams(use_tc_tiling_on_sc=False),
        scratch_types=[
            pltpu.VMEM((GPB, IDX_ROW), jnp.int32),
            pltpu.VMEM((ROWS_PER_BLOCK, HIDDEN), jnp.float32),
            pltpu.SemaphoreType.DMA,
        ],
    )
    def emb(ids_hbm, table_hbm, out_hbm, idx_v, rows_v, gsem):
        wid = lax.axis_index("s") * NC + lax.axis_index("c")
        id_row0 = wid * IDS_ROWS_PER_W
        out0 = wid * B_PER_W

        def block(g, carry):
            pltpu.sync_copy(
                ids_hbm.at[pl.ds(id_row0 + g * GPB, GPB)], idx_v
            )
            copies = []
            for j in range(GPB):
                copies.append(
                    pltpu.async_copy(
                        table_hbm.at[idx_v.at[j]],
                        rows_v.at[pl.ds(j * IDX_ROW, IDX_ROW)],
                        gsem,
                    )
                )
            for c in copies:
                c.wait()
            pltpu.sync_copy(
                rows_v,
                out_hbm.at[pl.ds(out0 + g * ROWS_PER_BLOCK, ROWS_PER_BLOCK)],
            )
            return carry

        lax.fori_loop(0, NBLK, block, 0)

    out = emb(ids2d, embed)
    return out.reshape(BATCH, SEQ, HIDDEN)


# speedup vs baseline: 1.0409x; 1.0409x over previous
import functools

import jax
import jax.numpy as jnp
from jax import lax
from jax.experimental import pallas as pl
from jax.experimental.pallas import tpu as pltpu
from jax.experimental.pallas import tpu_sc as plsc

BATCH = 4096
SEQ = 200
HIDDEN = 64
B = BATCH * SEQ
NC = 2
NS = 16
NW = NC * NS
B_PER_W = B // NW
IDX_ROW = 128
ROWS_PER_BLOCK = 512
GPB = ROWS_PER_BLOCK // IDX_ROW
NBLK = B_PER_W // ROWS_PER_BLOCK
IDS_ROWS_PER_W = B_PER_W // IDX_ROW
NBUF = 2


def kernel(input_ids, embed):
    ids2d = input_ids.reshape(B // IDX_ROW, IDX_ROW).astype(jnp.int32)
    mesh = plsc.VectorSubcoreMesh(core_axis_name="c", subcore_axis_name="s")

    @functools.partial(
        pl.kernel,
        mesh=mesh,
        out_type=jax.ShapeDtypeStruct((B, HIDDEN), jnp.float32),
        compiler_params=pltpu.CompilerParams(use_tc_tiling_on_sc=False),
        scratch_types=[
            pltpu.VMEM((IDS_ROWS_PER_W, IDX_ROW), jnp.int32),
            pltpu.VMEM((NBUF, ROWS_PER_BLOCK, HIDDEN), jnp.float32),
            pltpu.SemaphoreType.DMA((NBUF,)),
            pltpu.SemaphoreType.DMA((NBUF,)),
        ],
    )
    def emb(ids_hbm, table_hbm, out_hbm, idx_v, rows_v, gsem, ssem):
        wid = lax.axis_index("s") * NC + lax.axis_index("c")
        id_row0 = wid * IDS_ROWS_PER_W
        out0 = wid * B_PER_W

        pltpu.sync_copy(ids_hbm.at[pl.ds(id_row0, IDS_ROWS_PER_W)], idx_v)

        def fire_gathers(g, b):
            for j in range(GPB):
                pltpu.async_copy(
                    table_hbm.at[idx_v.at[g * GPB + j]],
                    rows_v.at[b, pl.ds(j * IDX_ROW, IDX_ROW)],
                    gsem.at[b],
                )

        def drain_gathers(b):
            pltpu.make_async_copy(
                table_hbm.at[pl.ds(0, ROWS_PER_BLOCK)], rows_v.at[b],
                gsem.at[b],
            ).wait()

        def fire_store(g, b):
            pltpu.async_copy(
                rows_v.at[b],
                out_hbm.at[pl.ds(out0 + g * ROWS_PER_BLOCK, ROWS_PER_BLOCK)],
                ssem.at[b],
            )

        def drain_store(b):
            pltpu.make_async_copy(
                rows_v.at[b], out_hbm.at[pl.ds(out0, ROWS_PER_BLOCK)],
                ssem.at[b],
            ).wait()

        for g in range(NBUF - 1):
            fire_gathers(g, g % NBUF)

        def step(o, carry):
            for b in range(NBUF):
                s = o * NBUF + b
                drain_gathers(b)
                fire_store(s, b)
                pb = (b - 1) % NBUF
                fb = s + NBUF - 1

                @pl.when(fb < NBLK)
                def _fire():
                    @pl.when(fb >= NBUF)
                    def _wait_prev_store():
                        drain_store(pb)

                    fire_gathers(fb, pb)

            return carry

        lax.fori_loop(0, NBLK // NBUF, step, 0)

        for b in range(NBUF):
            drain_store(b)

    out = emb(ids2d, embed)
    return out.reshape(BATCH, SEQ, HIDDEN)


# speedup vs baseline: 1.1882x; 1.1415x over previous
import functools

import jax
import jax.numpy as jnp
from jax import lax
from jax.experimental import pallas as pl
from jax.experimental.pallas import tpu as pltpu
from jax.experimental.pallas import tpu_sc as plsc

BATCH = 4096
SEQ = 200
HIDDEN = 64
NC = 2
NS = 16
NW = NC * NS
BPW = BATCH // NW
L = 16
NBUF = 2
OUT5 = (SEQ, HIDDEN // 8, BATCH // 128, 8, 128)


def kernel(input_ids, embed):
    ids_t = input_ids.T
    mesh = plsc.VectorSubcoreMesh(core_axis_name="c", subcore_axis_name="s")

    @functools.partial(
        pl.kernel,
        mesh=mesh,
        out_type=jax.ShapeDtypeStruct(OUT5, jnp.float32),
        compiler_params=pltpu.CompilerParams(
            use_tc_tiling_on_sc=False, needs_layout_passes=False
        ),
        scratch_types=[
            pltpu.VMEM((SEQ, BPW), jnp.int32),
            pltpu.VMEM((NBUF, BPW, HIDDEN), jnp.float32),
            pltpu.VMEM((NBUF, HIDDEN // 8, 8, BPW), jnp.float32),
            pltpu.SemaphoreType.DMA((NBUF,)),
            pltpu.SemaphoreType.DMA((NBUF,)),
        ],
    )
    def emb(ids_hbm, table_hbm, out_hbm, idsb, gbuf, slab, gsem, ssem):
        wid = lax.axis_index("s") * NC + lax.axis_index("c")
        bw = wid * BPW

        pltpu.sync_copy(ids_hbm.at[:, pl.ds(bw, BPW)], idsb)

        def fire_gather(s, b):
            pltpu.async_copy(
                table_hbm.at[idsb.at[s]], gbuf.at[b], gsem.at[b]
            )

        def drain_gather(b):
            pltpu.make_async_copy(
                table_hbm.at[pl.ds(0, BPW)], gbuf.at[b], gsem.at[b]
            ).wait()

        def fire_store(s, b):
            pltpu.async_copy(
                slab.at[b], out_hbm.at[s, :, wid], ssem.at[b]
            )

        def drain_store(b):
            pltpu.make_async_copy(
                slab.at[b], out_hbm.at[0, :, wid], ssem.at[b]
            ).wait()

        def transpose_block(s, b):
            iota = lax.iota(jnp.int32, L)
            rows = [iota + (i0 * L) for i0 in range(BPW // L)]

            def h0_body(hq, carry):
                h0 = hq * L
                for k in range(L):
                    hvec = ((iota + k) & (L - 1)) + h0
                    hr = hvec >> 3
                    hi = hvec & 7
                    for i0 in range(BPW // L):
                        v = plsc.load_gather(gbuf.at[b], [rows[i0], hvec])
                        plsc.store_scatter(
                            slab.at[b], [hr, hi, rows[i0]], v
                        )
                return carry

            lax.fori_loop(0, HIDDEN // L, h0_body, 0)

        fire_gather(0, 0)

        def step(o, carry):
            for b in range(NBUF):
                s = o * NBUF + b
                drain_gather(b)

                @pl.when(o >= 1)
                def _wait_slab():
                    drain_store(b)

                if b == 0:
                    fire_gather(s + 1, 1)
                else:

                    @pl.when(o < SEQ // NBUF - 1)
                    def _fire_next():
                        fire_gather(s + 1, 0)

                transpose_block(s, b)
                fire_store(s, b)
            return carry

        lax.fori_loop(0, SEQ // NBUF, step, 0)

        for b in range(NBUF):
            drain_store(b)

    out5 = emb(ids_t, embed)
    return out5.transpose(2, 4, 0, 1, 3).reshape(BATCH, SEQ, HIDDEN)


# speedup vs baseline: 1.3836x; 1.1644x over previous
import functools

import jax
import jax.numpy as jnp
from jax import lax
from jax.experimental import pallas as pl
from jax.experimental.pallas import tpu as pltpu
from jax.experimental.pallas import tpu_sc as plsc

BATCH = 4096
SEQ = 200
HIDDEN = 64
PADH = 128
B = BATCH * SEQ
NC = 2
NS = 16
NW = NC * NS
B_PER_W = B // NW
IDX_ROW = 128
ROWS_PER_BLOCK = 512
GPB = ROWS_PER_BLOCK // IDX_ROW
NBLK = B_PER_W // ROWS_PER_BLOCK
IDS_ROWS_PER_W = B_PER_W // IDX_ROW
NBUF = 2


def kernel(input_ids, embed):
    ids2d = input_ids.reshape(B // IDX_ROW, IDX_ROW)
    mesh = plsc.VectorSubcoreMesh(core_axis_name="c", subcore_axis_name="s")

    @functools.partial(
        pl.kernel,
        mesh=mesh,
        out_type=jax.ShapeDtypeStruct((B, PADH), jnp.float32),
        compiler_params=pltpu.CompilerParams(
            use_tc_tiling_on_sc=False, needs_layout_passes=False
        ),
        scratch_types=[
            pltpu.VMEM((IDS_ROWS_PER_W, IDX_ROW), jnp.int32),
            pltpu.VMEM((NBUF, ROWS_PER_BLOCK, HIDDEN), jnp.float32),
            pltpu.SemaphoreType.DMA((NBUF,)),
            pltpu.SemaphoreType.DMA((NBUF,)),
        ],
    )
    def emb(ids_hbm, table_hbm, out_hbm, idx_v, rows_v, gsem, ssem):
        wid = lax.axis_index("s") * NC + lax.axis_index("c")
        id_row0 = wid * IDS_ROWS_PER_W
        out0 = wid * B_PER_W

        pltpu.sync_copy(ids_hbm.at[pl.ds(id_row0, IDS_ROWS_PER_W)], idx_v)

        def fire_gathers(g, b):
            for j in range(GPB):
                pltpu.async_copy(
                    table_hbm.at[idx_v.at[g * GPB + j]],
                    rows_v.at[b, pl.ds(j * IDX_ROW, IDX_ROW)],
                    gsem.at[b],
                )

        def drain_gathers(b):
            pltpu.make_async_copy(
                table_hbm.at[pl.ds(0, ROWS_PER_BLOCK)], rows_v.at[b],
                gsem.at[b],
            ).wait()

        def fire_store(g, b):
            pltpu.async_copy(
                rows_v.at[b],
                out_hbm.at[
                    pl.ds(out0 + g * ROWS_PER_BLOCK, ROWS_PER_BLOCK),
                    pl.ds(0, HIDDEN),
                ],
                ssem.at[b],
            )

        def drain_store(b):
            pltpu.make_async_copy(
                rows_v.at[b],
                out_hbm.at[pl.ds(out0, ROWS_PER_BLOCK), pl.ds(0, HIDDEN)],
                ssem.at[b],
            ).wait()

        for g in range(NBUF - 1):
            fire_gathers(g, g % NBUF)

        def step(o, carry):
            for b in range(NBUF):
                s = o * NBUF + b
                drain_gathers(b)
                fire_store(s, b)
                pb = (b - 1) % NBUF
                fb = s + NBUF - 1

                @pl.when(fb < NBLK)
                def _fire():
                    @pl.when(fb >= NBUF)
                    def _wait_prev_store():
                        drain_store(pb)

                    fire_gathers(fb, pb)

            return carry

        lax.fori_loop(0, NBLK // NBUF, step, 0)

        for b in range(NBUF):
            drain_store(b)

    out = emb(ids2d, embed)
    return out[:, :HIDDEN].reshape(BATCH, SEQ, HIDDEN)


# speedup vs baseline: 1.3885x; 1.0036x over previous
import functools

import jax
import jax.numpy as jnp
from jax import lax
from jax.experimental import pallas as pl
from jax.experimental.pallas import tpu as pltpu
from jax.experimental.pallas import tpu_sc as plsc

BATCH = 4096
SEQ = 200
HIDDEN = 64
PADH = 128
B = BATCH * SEQ
NC = 2
NS = 16
NW = NC * NS
B_PER_W = B // NW
IDX_ROW = 128
ROWS_PER_BLOCK = 256
GPB = ROWS_PER_BLOCK // IDX_ROW
NBLK = B_PER_W // ROWS_PER_BLOCK
IDS_ROWS_PER_W = B_PER_W // IDX_ROW
NBUF = 4


def kernel(input_ids, embed):
    ids2d = input_ids.reshape(B // IDX_ROW, IDX_ROW)
    mesh = plsc.VectorSubcoreMesh(core_axis_name="c", subcore_axis_name="s")

    @functools.partial(
        pl.kernel,
        mesh=mesh,
        out_type=jax.ShapeDtypeStruct((B, PADH), jnp.float32),
        compiler_params=pltpu.CompilerParams(
            use_tc_tiling_on_sc=False, needs_layout_passes=False
        ),
        scratch_types=[
            pltpu.VMEM((IDS_ROWS_PER_W, IDX_ROW), jnp.int32),
            pltpu.VMEM((NBUF, ROWS_PER_BLOCK, HIDDEN), jnp.float32),
            pltpu.SemaphoreType.DMA((NBUF,)),
            pltpu.SemaphoreType.DMA((NBUF,)),
        ],
    )
    def emb(ids_hbm, table_hbm, out_hbm, idx_v, rows_v, gsem, ssem):
        wid = lax.axis_index("s") * NC + lax.axis_index("c")
        id_row0 = wid * IDS_ROWS_PER_W
        out0 = wid * B_PER_W

        pltpu.sync_copy(ids_hbm.at[pl.ds(id_row0, IDS_ROWS_PER_W)], idx_v)

        def fire_gathers(g, b):
            for j in range(GPB):
                pltpu.async_copy(
                    table_hbm.at[idx_v.at[g * GPB + j]],
                    rows_v.at[b, pl.ds(j * IDX_ROW, IDX_ROW)],
                    gsem.at[b],
                )

        def drain_gathers(b):
            pltpu.make_async_copy(
                table_hbm.at[pl.ds(0, ROWS_PER_BLOCK)], rows_v.at[b],
                gsem.at[b],
            ).wait()

        def fire_store(g, b):
            pltpu.async_copy(
                rows_v.at[b],
                out_hbm.at[
                    pl.ds(out0 + g * ROWS_PER_BLOCK, ROWS_PER_BLOCK),
                    pl.ds(0, HIDDEN),
                ],
                ssem.at[b],
            )

        def drain_store(b):
            pltpu.make_async_copy(
                rows_v.at[b],
                out_hbm.at[pl.ds(out0, ROWS_PER_BLOCK), pl.ds(0, HIDDEN)],
                ssem.at[b],
            ).wait()

        for g in range(NBUF - 1):
            fire_gathers(g, g % NBUF)

        def step(o, carry):
            for b in range(NBUF):
                s = o * NBUF + b
                drain_gathers(b)
                fire_store(s, b)
                pb = (b - 1) % NBUF
                fb = s + NBUF - 1

                @pl.when(fb < NBLK)
                def _fire():
                    @pl.when(fb >= NBUF)
                    def _wait_prev_store():
                        drain_store(pb)

                    fire_gathers(fb, pb)

            return carry

        lax.fori_loop(0, NBLK // NBUF, step, 0)

        for b in range(NBUF):
            drain_store(b)

    out = emb(ids2d, embed)
    return out[:, :HIDDEN].reshape(BATCH, SEQ, HIDDEN)
